# baseline (speedup 1.0000x reference)
import jax
import jax.numpy as jnp
from jax.experimental import pallas as pl
from jax.experimental.pallas import tpu as pltpu

_P = 4194304
_COLS = 2048
_ROWS = _P // _COLS
_BLK_ROWS = 256
_NCHUNK = _ROWS // _BLK_ROWS


def _scale_body(a_smem, b_vmem, out_vmem):
    b = pl.program_id(0)
    out_vmem[...] = a_smem[b] * b_vmem[...]


def kernel(B, A):
    b3 = B.reshape(2, _ROWS, _COLS)
    a2 = A.reshape(2)
    out = pl.pallas_call(
        _scale_body,
        grid=(2, _NCHUNK),
        in_specs=[
            pl.BlockSpec(memory_space=pltpu.SMEM),
            pl.BlockSpec((1, _BLK_ROWS, _COLS), lambda b, j: (b, j, 0)),
        ],
        out_specs=pl.BlockSpec((1, _BLK_ROWS, _COLS), lambda b, j: (b, j, 0)),
        out_shape=jax.ShapeDtypeStruct((2, _ROWS, _COLS), jnp.float32),
    )(a2, b3)
    return out.reshape(2, 1, _P)


# speedup vs baseline: 1.0345x; 1.0345x over previous
import jax
import jax.numpy as jnp
from jax.experimental import pallas as pl
from jax.experimental.pallas import tpu as pltpu

_P = 4194304
_COLS = 2048
_ROWS = _P // _COLS
_BLK_ROWS = 256
_NCHUNK_PER_B = _ROWS // _BLK_ROWS
_NCHUNK = 2 * _NCHUNK_PER_B
_NBUF = 4


def _body(a_smem, b_hbm, out_hbm, inb, outb, in_sems, out_sems):
    nz = [a_smem[0] != 0.0, a_smem[1] != 0.0]
    av = [a_smem[0], a_smem[1]]

    def in_copy(i):
        b, j = divmod(i, _NCHUNK_PER_B)
        return pltpu.make_async_copy(
            b_hbm.at[b, pl.ds(j * _BLK_ROWS, _BLK_ROWS), :],
            inb.at[i % _NBUF],
            in_sems.at[i % _NBUF],
        ), nz[b]

    def out_copy(i):
        b, j = divmod(i, _NCHUNK_PER_B)
        return pltpu.make_async_copy(
            outb.at[i % _NBUF],
            out_hbm.at[b, pl.ds(j * _BLK_ROWS, _BLK_ROWS), :],
            out_sems.at[i % _NBUF],
        )

    def start_in(i):
        cp, p = in_copy(i)

        @pl.when(p)
        def _():
            cp.start()

    for k in range(_NBUF - 1):
        start_in(k)

    for i in range(_NCHUNK):
        if i + _NBUF - 1 < _NCHUNK:
            start_in(i + _NBUF - 1)
        b = i // _NCHUNK_PER_B
        s = i % _NBUF
        cp_in, p = in_copy(i)
        if i >= _NBUF:
            out_copy(i - _NBUF).wait()

        @pl.when(p)
        def _():
            cp_in.wait()
            outb[s] = av[b] * inb[s]

        @pl.when(jnp.logical_not(p))
        def _():
            outb[s] = jnp.zeros((_BLK_ROWS, _COLS), jnp.float32)

        out_copy(i).start()

    for i in range(_NCHUNK - _NBUF, _NCHUNK):
        out_copy(i).wait()


def kernel(B, A):
    b3 = B.reshape(2, _ROWS, _COLS)
    a2 = A.reshape(2)
    out = pl.pallas_call(
        _body,
        in_specs=[
            pl.BlockSpec(memory_space=pltpu.SMEM),
            pl.BlockSpec(memory_space=pl.ANY),
        ],
        out_specs=pl.BlockSpec(memory_space=pl.ANY),
        out_shape=jax.ShapeDtypeStruct((2, _ROWS, _COLS), jnp.float32),
        scratch_shapes=[
            pltpu.VMEM((_NBUF, _BLK_ROWS, _COLS), jnp.float32),
            pltpu.VMEM((_NBUF, _BLK_ROWS, _COLS), jnp.float32),
            pltpu.SemaphoreType.DMA((_NBUF,)),
            pltpu.SemaphoreType.DMA((_NBUF,)),
        ],
    )(a2, b3)
    return out.reshape(2, 1, _P)


# speedup vs baseline: 9.7922x; 9.4657x over previous
import jax
import jax.numpy as jnp
from jax.experimental import pallas as pl
from jax.experimental.pallas import tpu as pltpu

_P = 4194304
_CHUNK = 1 << 19
_NCHUNK = _P // _CHUNK


def _scale_body(a_smem, b_vmem, out_vmem):
    b = pl.program_id(0)
    out_vmem[...] = a_smem[b] * b_vmem[...]


def kernel(B, A):
    a2 = A.reshape(2)
    out = pl.pallas_call(
        _scale_body,
        grid=(2, _NCHUNK),
        in_specs=[
            pl.BlockSpec(memory_space=pltpu.SMEM),
            pl.BlockSpec((1, 1, _CHUNK), lambda b, j: (b, 0, j)),
        ],
        out_specs=pl.BlockSpec((1, 1, _CHUNK), lambda b, j: (b, 0, j)),
        out_shape=jax.ShapeDtypeStruct((2, 1, _P), jnp.float32),
    )(a2, B)
    return out


# speedup vs baseline: 12.0915x; 1.2348x over previous
import jax
import jax.numpy as jnp
from jax.experimental import pallas as pl
from jax.experimental.pallas import tpu as pltpu

_P = 4194304
_CHUNK = 1 << 19
_NCHUNK = _P // _CHUNK
_TOTAL = 2 * _NCHUNK


def _body(a_smem, b_any, out_vmem, inb, sems):
    bi = pl.program_id(0)
    j = pl.program_id(1)
    i = bi * _NCHUNK + j

    def in_copy(b_idx, j_idx, slot):
        return pltpu.make_async_copy(
            b_any.at[b_idx, pl.ds(0, 1), pl.ds(j_idx * _CHUNK, _CHUNK)],
            inb.at[slot],
            sems.at[slot],
        )

    @pl.when(i == 0)
    def _():
        @pl.when(a_smem[0] != 0.0)
        def _():
            in_copy(0, 0, 0).start()

    i1 = i + 1
    b1 = jnp.minimum(i1 // _NCHUNK, 1)
    j1 = i1 % _NCHUNK

    @pl.when(jnp.logical_and(i1 < _TOTAL, a_smem[b1] != 0.0))
    def _():
        in_copy(b1, j1, i1 % 2).start()

    a = a_smem[bi]

    @pl.when(a != 0.0)
    def _():
        in_copy(bi, j, i % 2).wait()
        out_vmem[0] = a * inb[i % 2]

    @pl.when(a == 0.0)
    def _():
        out_vmem[0] = jnp.zeros((1, _CHUNK), jnp.float32)


def kernel(B, A):
    a2 = A.reshape(2)
    out = pl.pallas_call(
        _body,
        grid=(2, _NCHUNK),
        in_specs=[
            pl.BlockSpec(memory_space=pltpu.SMEM),
            pl.BlockSpec(memory_space=pl.ANY),
        ],
        out_specs=pl.BlockSpec((1, 1, _CHUNK), lambda b, j: (b, 0, j)),
        out_shape=jax.ShapeDtypeStruct((2, 1, _P), jnp.float32),
        scratch_shapes=[
            pltpu.VMEM((2, 1, _CHUNK), jnp.float32),
            pltpu.SemaphoreType.DMA((2,)),
        ],
    )(a2, B)
    return out
